# Initial kernel scaffold; baseline (speedup 1.0000x reference)
import jax
import jax.numpy as jnp
from jax.experimental import pallas as pl


def kernel(x, table, W, b):
    raise NotImplementedError("write your pallas kernel here")


# speedup vs baseline: 14.0829x; 14.0829x over previous
import functools

import jax
import jax.numpy as jnp
from jax import lax
from jax.experimental import pallas as pl
from jax.experimental.pallas import tpu as pltpu
from jax.experimental.pallas import tpu_sc as plsc

B = 16384
FW = 20
ED = 32
NC = 2
NS = 16
NW = NC * NS
RPW = B // NW
CHUNK = 32
NCH = RPW // CHUNK
K = CHUNK * FW // 128


def _perm(v, idx16):
    return lax.gather(
        v, idx16.reshape(16, 1),
        dimension_numbers=lax.GatherDimensionNumbers(
            offset_dims=(), collapsed_slice_dims=(0,), start_index_map=(0,)),
        slice_sizes=(1,),
        mode=lax.GatherScatterMode.PROMISE_IN_BOUNDS)


def _sc_body(table_hbm, xidx_hbm, w_hbm, b_hbm, out_hbm,
             idx_v, rows_v, w_v, b_v, out_v, sem):
    wid = lax.axis_index("s") * NC + lax.axis_index("c")

    pltpu.sync_copy(w_hbm, w_v)
    pltpu.sync_copy(b_hbm, b_v)
    wv = [w_v[i, :] for i in range(2 * FW)]
    bv = b_v[:]
    lanes = lax.iota(jnp.int32, 16)
    lane_masks = [lanes == r for r in range(16)]
    bfly = [lanes ^ off for off in (1, 2, 4, 8)]

    def chunk_body(c, carry):
        pltpu.sync_copy(xidx_hbm.at[wid, c], idx_v)
        cps = [
            pltpu.async_copy(
                table_hbm.at[idx_v.at[j]],
                rows_v.at[pl.ds(j * 128, 128), :],
                sem,
            )
            for j in range(K)
        ]
        for cp in cps:
            cp.wait()
        for h in range(CHUNK // 16):
            res = jnp.zeros((16,), jnp.float32)
            for r in range(16):
                g0 = (h * 16 + r) * FW
                acc0 = rows_v[g0, pl.ds(0, 16)] * wv[0]
                acc1 = rows_v[g0, pl.ds(16, 16)] * wv[1]
                for f in range(1, FW):
                    acc0 = acc0 + rows_v[g0 + f, pl.ds(0, 16)] * wv[2 * f]
                    acc1 = acc1 + rows_v[g0 + f, pl.ds(16, 16)] * wv[2 * f + 1]
                tot = acc0 + acc1
                for pm in bfly:
                    tot = tot + _perm(tot, pm)
                res = jnp.where(lane_masks[r], tot, res)
            out_v[pl.ds(c * CHUNK + h * 16, 16)] = jnp.exp(res + bv)
        return carry

    lax.fori_loop(0, NCH, chunk_body, 0)
    pltpu.sync_copy(out_v, out_hbm.at[pl.ds(wid * RPW, RPW)])


@jax.jit
def _run(table, xi, w2, b16):
    mesh = plsc.VectorSubcoreMesh(core_axis_name="c", subcore_axis_name="s")
    return pl.kernel(
        _sc_body,
        mesh=mesh,
        out_type=jax.ShapeDtypeStruct((B,), jnp.float32),
        compiler_params=pltpu.CompilerParams(use_tc_tiling_on_sc=False),
        scratch_types=[
            pltpu.VMEM((K, 128), jnp.int32),
            pltpu.VMEM((CHUNK * FW, ED), jnp.float32),
            pltpu.VMEM((2 * FW, 16), jnp.float32),
            pltpu.VMEM((16,), jnp.float32),
            pltpu.VMEM((RPW,), jnp.float32),
            pltpu.SemaphoreType.DMA,
        ],
    )(table, xi, w2, b16)


def kernel(x, table, W, b):
    xi = x.astype(jnp.int32).reshape(NW, NCH, K, 128)
    w2 = W.astype(jnp.float32).reshape(2 * FW, 16)
    b16 = jnp.broadcast_to(b.astype(jnp.float32), (16,))
    return _run(table, xi, w2, b16)
